# baseline (speedup 1.0000x reference)
import functools

import jax
import jax.numpy as jnp
from jax import lax
from jax.experimental import pallas as pl
from jax.experimental.pallas import tpu as pltpu
from jax.experimental.pallas import tpu_sc as plsc

_B = 16384
_E = 64
_NCTX = 5
_NC = 2
_NS = 16
_NW = _NC * _NS
_BPW = _B // _NW
_C = 128
_NCHUNK = _BPW // _C
_DPC = _C * _NCTX
_VH = 500000


def _sc_dots(tgt_h, tgt_p, ctx_h, ctx_p, tt2, ct2):
    mesh = plsc.VectorSubcoreMesh(core_axis_name="c", subcore_axis_name="s")

    @functools.partial(
        pl.kernel,
        mesh=mesh,
        compiler_params=pltpu.CompilerParams(
            needs_layout_passes=False, use_tc_tiling_on_sc=False),
        out_type=jax.ShapeDtypeStruct((_B * _NCTX,), jnp.float32),
        scratch_types=[
            pltpu.VMEM((_C,), jnp.int32),
            pltpu.VMEM((_C,), jnp.int32),
            pltpu.VMEM((_NCTX, _C), jnp.int32),
            pltpu.VMEM((_DPC,), jnp.int32),
            pltpu.VMEM((_C, 2 * _E), jnp.float32),
            pltpu.VMEM((_DPC, 2 * _E), jnp.float32),
            pltpu.VMEM((_DPC,), jnp.float32),
            pltpu.SemaphoreType.DMA,
        ],
    )
    def body(th_hbm, tp_hbm, ch_hbm, cp_hbm, tt_hbm, ct_hbm, out_hbm,
             ti_v, tp_v, ci_v, cp_v, trow_v, crow_v, out_v, sem):
        wid = lax.axis_index("s") * _NC + lax.axis_index("c")
        iota = lax.iota(jnp.int32, 16)
        mask15 = iota == 15

        for chunk in range(_NCHUNK):
            base = wid * _BPW + chunk * _C
            pltpu.sync_copy(th_hbm.at[pl.ds(base, _C)], ti_v)
            pltpu.sync_copy(tp_hbm.at[pl.ds(base, _C)], tp_v)
            pltpu.sync_copy(cp_hbm.at[pl.ds(base * _NCTX, _DPC)], cp_v)
            for k in range(_NCTX):
                pltpu.sync_copy(
                    ch_hbm.at[pl.ds(base * _NCTX + k * _C, _C)],
                    ci_v.at[k])

            copies = [pltpu.async_copy(tt_hbm.at[ti_v], trow_v, sem)]
            for k in range(_NCTX):
                copies.append(
                    pltpu.async_copy(ct_hbm.at[ci_v.at[k]],
                                     crow_v.at[pl.ds(k * _C, _C)], sem))
            for c in copies:
                c.wait()

            def block_body(blk, _):
                tpv = tp_v[pl.ds(blk * 16, 16)]
                cpv = [cp_v[pl.ds(blk * 80 + g * 16, 16)] for g in range(5)]
                for ii in range(16):
                    i = blk * 16 + ii
                    toff = tpv[ii] * _E
                    t = [trow_v[i, pl.ds(toff + 16 * k, 16)]
                         for k in range(4)]
                    for jj in range(_NCTX):
                        d = ii * _NCTX + jj
                        pos = blk * 80 + d
                        coff = cpv[d // 16][d % 16] * _E
                        p = t[0] * crow_v[pos, pl.ds(coff, 16)]
                        p = p + t[1] * crow_v[pos, pl.ds(coff + 16, 16)]
                        p = p + t[2] * crow_v[pos, pl.ds(coff + 32, 16)]
                        p = p + t[3] * crow_v[pos, pl.ds(coff + 48, 16)]
                        s = plsc.cumsum(p)
                        idxv = jnp.full((16,), pos, jnp.int32)
                        plsc.store_scatter(out_v, [idxv], s, mask=mask15)
                return 0

            lax.fori_loop(0, _C // 16, block_body, 0)
            pltpu.sync_copy(out_v, out_hbm.at[pl.ds(base * _NCTX, _DPC)])

    return body(tgt_h, tgt_p, ctx_h, ctx_p, tt2, ct2)


def kernel(target, context, target_table, context_table):
    tgt_idx = target.reshape(-1).astype(jnp.int32)
    ctx_idx = context.reshape(-1).astype(jnp.int32)
    tgt_h = tgt_idx >> 1
    tgt_p = tgt_idx & 1
    ctx_h = ctx_idx >> 1
    ctx_p = ctx_idx & 1
    tt2 = target_table.reshape(_VH, 2 * _E)
    ct2 = context_table.reshape(_VH, 2 * _E)
    out = _sc_dots(tgt_h, tgt_p, ctx_h, ctx_p, tt2, ct2)
    return out.reshape(_B, _NCTX)
